# Initial kernel scaffold; baseline (speedup 1.0000x reference)
import jax
import jax.numpy as jnp
from jax.experimental import pallas as pl


def kernel(x, edge_index, W, b):
    raise NotImplementedError("write your pallas kernel here")


# speedup vs baseline: 2.4294x; 2.4294x over previous
import functools

import jax
import jax.numpy as jnp
from jax import lax
from jax.experimental import pallas as pl
from jax.experimental.pallas import tpu as pltpu
from jax.experimental.pallas import tpu_sc as plsc

N = 10000
C = 128
O = 128
K = 32

NC = 2
NS = 16
NW = NC * NS
NPW = 320
NPAD = NW * NPW
G = 2
R = G * K
NG = NPW // G
LANES = 16
NCH = O // LANES


def _mm_body(x_ref, wa_ref, wb_ref, a_ref, b_ref):
    xb = x_ref[...]
    a_ref[...] = jnp.dot(xb, wa_ref[...], preferred_element_type=jnp.float32)
    b_ref[...] = jnp.dot(xb, wb_ref[...], preferred_element_type=jnp.float32)


def _node_tables(xt, wa, wb):
    blk = 2000
    grid = (N // blk,)
    return pl.pallas_call(
        _mm_body,
        grid=grid,
        in_specs=[
            pl.BlockSpec((blk, C), lambda i: (i, 0)),
            pl.BlockSpec((C, O), lambda i: (0, 0)),
            pl.BlockSpec((C, O), lambda i: (0, 0)),
        ],
        out_specs=[
            pl.BlockSpec((blk, O), lambda i: (i, 0)),
            pl.BlockSpec((blk, O), lambda i: (i, 0)),
        ],
        out_shape=[
            jax.ShapeDtypeStruct((N, O), jnp.float32),
            jax.ShapeDtypeStruct((N, O), jnp.float32),
        ],
    )(xt, wa, wb)


def _tree_max(vs):
    while len(vs) > 1:
        nxt = [jnp.maximum(vs[i], vs[i + 1]) for i in range(0, len(vs) - 1, 2)]
        if len(vs) % 2:
            nxt.append(vs[-1])
        vs = nxt
    return vs[0]


def _sc_body(a_hbm, b_hbm, idx1_hbm, idx0_hbm, bias_hbm, out_hbm,
             idx1_v, idx0_v, bias_v, out_v,
             buf_a0, buf_b0, buf_a1, buf_b1,
             sem_a0, sem_b0, sem_a1, sem_b1):
    wid = lax.axis_index("s") * NC + lax.axis_index("c")

    pltpu.sync_copy(idx1_hbm.at[wid], idx1_v)
    pltpu.sync_copy(idx0_hbm.at[wid], idx0_v)
    pltpu.sync_copy(bias_hbm, bias_v)

    def start(g, idx_v, table, buf, sem):
        return pltpu.async_copy(table.at[idx_v.at[g]], buf, sem)

    def wait(g, idx_v, table, buf, sem):
        pltpu.make_async_copy(table.at[idx_v.at[g]], buf, sem).wait()

    def compute(g, buf_a, buf_b):
        for j in range(G):
            base = j * K
            for c in range(NCH):
                sl = pl.ds(c * LANES, LANES)
                vs = [buf_a[base + r, sl] + buf_b[base + r, sl]
                      for r in range(K)]
                m = _tree_max(vs)
                m = jnp.maximum(m + bias_v[sl], 0.0)
                out_v[pl.ds((g * G + j) * O + c * LANES, LANES)] = m

    start(0, idx1_v, a_hbm, buf_a0, sem_a0)
    start(0, idx0_v, b_hbm, buf_b0, sem_b0)

    def body(g2, carry):
        g0 = 2 * g2
        g1 = g0 + 1
        start(g1, idx1_v, a_hbm, buf_a1, sem_a1)
        start(g1, idx0_v, b_hbm, buf_b1, sem_b1)
        wait(g0, idx1_v, a_hbm, buf_a0, sem_a0)
        wait(g0, idx0_v, b_hbm, buf_b0, sem_b0)
        compute(g0, buf_a0, buf_b0)
        start(g0 + 2, idx1_v, a_hbm, buf_a0, sem_a0)
        start(g0 + 2, idx0_v, b_hbm, buf_b0, sem_b0)
        wait(g1, idx1_v, a_hbm, buf_a1, sem_a1)
        wait(g1, idx0_v, b_hbm, buf_b1, sem_b1)
        compute(g1, buf_a1, buf_b1)
        return carry

    lax.fori_loop(0, NG // 2, body, 0)

    wait(NG, idx1_v, a_hbm, buf_a0, sem_a0)
    wait(NG, idx0_v, b_hbm, buf_b0, sem_b0)

    pltpu.sync_copy(out_v, out_hbm.at[wid])


@functools.partial(
    pl.kernel,
    out_type=jax.ShapeDtypeStruct((NW, NPW * O), jnp.float32),
    mesh=plsc.VectorSubcoreMesh(core_axis_name="c", subcore_axis_name="s"),
    scratch_types=[
        pltpu.VMEM((NG + 1, R), jnp.int32),
        pltpu.VMEM((NG + 1, R), jnp.int32),
        pltpu.VMEM((O,), jnp.float32),
        pltpu.VMEM((NPW * O,), jnp.float32),
        pltpu.VMEM((R, O), jnp.float32),
        pltpu.VMEM((R, O), jnp.float32),
        pltpu.VMEM((R, O), jnp.float32),
        pltpu.VMEM((R, O), jnp.float32),
        pltpu.SemaphoreType.DMA,
        pltpu.SemaphoreType.DMA,
        pltpu.SemaphoreType.DMA,
        pltpu.SemaphoreType.DMA,
    ],
)
def _sc_gather_max(a_hbm, b_hbm, idx1_hbm, idx0_hbm, bias_hbm, out_hbm,
                   *rest):
    _sc_body(a_hbm, b_hbm, idx1_hbm, idx0_hbm, bias_hbm, out_hbm, *rest)


def kernel(x, edge_index, W, b):
    xt = x[0, :, :, 0].T
    w1 = W[:, :C]
    w2 = W[:, C:]
    wa = (w1 - w2).T
    wb = w2.T

    a_tab, b_tab = _node_tables(xt, wa, wb)

    ei = edge_index.astype(jnp.int32).reshape(2, N * K)
    pad = NPAD * K - N * K
    idx1 = jnp.pad(ei[1], (0, pad)).reshape(NW, NG, R)
    idx0 = jnp.pad(ei[0], (0, pad)).reshape(NW, NG, R)
    zrow = jnp.zeros((NW, 1, R), jnp.int32)
    idx1 = jnp.concatenate([idx1, zrow], axis=1)
    idx0 = jnp.concatenate([idx0, zrow], axis=1)

    out = _sc_gather_max(a_tab, b_tab, idx1, idx0, b)
    out = out.reshape(NPAD, O)[:N].T
    return out[None]
